# baseline (speedup 1.0000x reference)
import functools

import jax
import jax.numpy as jnp
from jax import lax
from jax.experimental import pallas as pl
from jax.experimental.pallas import tpu as pltpu

BT = 256


def _routing(y, num_experts, num_blocks):
    m = y.shape[0]
    y32 = y.astype(jnp.int32)
    onehot = (y32[:, None] == jnp.arange(num_experts, dtype=jnp.int32)[None, :])
    counts = jnp.sum(onehot.astype(jnp.int32), axis=0)
    padded = ((counts + BT - 1) // BT) * BT
    ends = jnp.cumsum(padded)
    off = ends - padded
    rank = jnp.cumsum(onehot.astype(jnp.int32), axis=0) - 1
    r = jnp.take_along_axis(rank, y32[:, None], axis=1)[:, 0]
    pos = off[y32] + r
    total = num_blocks * BT
    perm = jnp.zeros((total,), jnp.int32).at[pos].set(
        jnp.arange(m, dtype=jnp.int32))
    block_starts = jnp.arange(num_blocks, dtype=jnp.int32) * BT
    block_expert = jnp.sum(
        (block_starts[:, None] >= ends[None, :]).astype(jnp.int32), axis=1)
    block_expert = jnp.minimum(block_expert, num_experts - 1)
    return perm, pos, block_expert


def _gmm_body(nbk, act, e_ref, x_ref, w_ref, b_ref, o_ref, acc_ref):
    kk = pl.program_id(2)

    @pl.when(kk == 0)
    def _():
        acc_ref[...] = jnp.zeros_like(acc_ref)

    acc_ref[...] += jnp.dot(x_ref[...], w_ref[0],
                            preferred_element_type=jnp.float32)

    @pl.when(kk == nbk - 1)
    def _():
        v = acc_ref[...] + b_ref[0, 0]
        o_ref[...] = act(v)


def _gmm(xs, w, b, block_expert, bj, bk, act):
    p, din = xs.shape
    ne, _, dout = w.shape
    b3 = b.reshape(ne, 1, dout)
    nbt, nbj, nbk = p // BT, dout // bj, din // bk
    grid_spec = pltpu.PrefetchScalarGridSpec(
        num_scalar_prefetch=1,
        grid=(nbt, nbj, nbk),
        in_specs=[
            pl.BlockSpec((BT, bk), lambda i, j, k, e: (i, k)),
            pl.BlockSpec((1, bk, bj), lambda i, j, k, e: (e[i], k, j)),
            pl.BlockSpec((1, 1, bj), lambda i, j, k, e: (e[i], 0, j)),
        ],
        out_specs=pl.BlockSpec((BT, bj), lambda i, j, k, e: (i, j)),
        scratch_shapes=[pltpu.VMEM((BT, bj), jnp.float32)],
    )
    return pl.pallas_call(
        functools.partial(_gmm_body, nbk, act),
        grid_spec=grid_spec,
        out_shape=jax.ShapeDtypeStruct((p, dout), jnp.float32),
    )(block_expert, xs, w, b3)


def _gmm_final_body(nbk, nbj_half, e_ref, x_ref, w_ref, b_ref,
                    mu_ref, sig_ref, acc_ref):
    kk = pl.program_id(2)
    jj = pl.program_id(1)

    @pl.when(kk == 0)
    def _():
        acc_ref[...] = jnp.zeros_like(acc_ref)

    acc_ref[...] += jnp.dot(x_ref[...], w_ref[0],
                            preferred_element_type=jnp.float32)

    @pl.when((kk == nbk - 1) & (jj < nbj_half))
    def _():
        mu_ref[...] = acc_ref[...] + b_ref[0, 0]

    @pl.when((kk == nbk - 1) & (jj >= nbj_half))
    def _():
        v = acc_ref[...] + b_ref[0, 0]
        sig_ref[...] = jnp.maximum(v, 0.0) + jnp.log1p(jnp.exp(-jnp.abs(v)))


def _gmm_final(xs, w, b, block_expert, bj, bk):
    p, din = xs.shape
    ne, _, dout = w.shape
    b3 = b.reshape(ne, 1, dout)
    half = dout // 2
    nbt, nbj, nbk = p // BT, dout // bj, din // bk
    nbj_half = nbj // 2
    grid_spec = pltpu.PrefetchScalarGridSpec(
        num_scalar_prefetch=1,
        grid=(nbt, nbj, nbk),
        in_specs=[
            pl.BlockSpec((BT, bk), lambda i, j, k, e: (i, k)),
            pl.BlockSpec((1, bk, bj), lambda i, j, k, e: (e[i], k, j)),
            pl.BlockSpec((1, 1, bj), lambda i, j, k, e: (e[i], 0, j)),
        ],
        out_specs=[
            pl.BlockSpec((BT, bj),
                         lambda i, j, k, e: (i, jnp.minimum(j, nbj // 2 - 1))),
            pl.BlockSpec((BT, bj),
                         lambda i, j, k, e: (i, jnp.maximum(j - nbj // 2, 0))),
        ],
        scratch_shapes=[pltpu.VMEM((BT, bj), jnp.float32)],
    )
    return pl.pallas_call(
        functools.partial(_gmm_final_body, nbk, nbj_half),
        grid_spec=grid_spec,
        out_shape=[
            jax.ShapeDtypeStruct((p, half), jnp.float32),
            jax.ShapeDtypeStruct((p, half), jnp.float32),
        ],
    )(block_expert, xs, w, b3)


def kernel(x, W0, b0, W1, b1, W2, b2, y):
    m, din = x.shape
    num_experts = W0.shape[0]
    num_blocks = (m + num_experts * BT) // BT
    p = num_blocks * BT

    perm, pos, block_expert = _routing(y, num_experts, num_blocks)

    xs = x[perm]

    h0 = _gmm(xs, W0, b0, block_expert, bj=512, bk=512,
              act=lambda v: jnp.maximum(v, 0.0))
    h1 = _gmm(h0, W1, b1, block_expert, bj=512, bk=512,
              act=lambda v: jnp.maximum(v, 0.0))
    mu_s, sig_s = _gmm_final(h1, W2, b2, block_expert, bj=512, bk=512)

    mu = mu_s[pos]
    sigma = sig_s[pos]
    return (mu, sigma)
